# baseline (speedup 1.0000x reference)
import jax
import jax.numpy as jnp
from jax.experimental import pallas as pl

_BLK = 2048


def _mask_body(x_ref, m_ref, o_ref):
    o_ref[...] = jnp.where(m_ref[...] != 0, x_ref[...], jnp.float32(0))


def kernel(inputs, mask):
    b, u = inputs.shape
    m2 = mask.reshape(1, u).astype(jnp.int8)
    grid = (u // _BLK,)
    return pl.pallas_call(
        _mask_body,
        grid=grid,
        in_specs=[
            pl.BlockSpec((b, _BLK), lambda j: (0, j)),
            pl.BlockSpec((1, _BLK), lambda j: (0, j)),
        ],
        out_specs=pl.BlockSpec((b, _BLK), lambda j: (0, j)),
        out_shape=jax.ShapeDtypeStruct((b, u), inputs.dtype),
    )(inputs, m2)


# speedup vs baseline: 1.2843x; 1.2843x over previous
import jax
import jax.numpy as jnp
from jax.experimental import pallas as pl
from jax.experimental.pallas import tpu as pltpu

_BLK = 4096


def _mask_body(x_ref, m_ref, o_ref):
    o_ref[...] = jnp.where(m_ref[...] != 0, x_ref[...], jnp.float32(0))


def kernel(inputs, mask):
    b, u = inputs.shape
    m2 = mask.reshape(1, u).astype(jnp.int8)
    grid = (u // _BLK,)
    return pl.pallas_call(
        _mask_body,
        grid=grid,
        in_specs=[
            pl.BlockSpec((b, _BLK), lambda j: (0, j)),
            pl.BlockSpec((1, _BLK), lambda j: (0, j)),
        ],
        out_specs=pl.BlockSpec((b, _BLK), lambda j: (0, j)),
        out_shape=jax.ShapeDtypeStruct((b, u), inputs.dtype),
        compiler_params=pltpu.CompilerParams(
            dimension_semantics=("parallel",),
        ),
    )(inputs, m2)
